# Initial kernel scaffold; baseline (speedup 1.0000x reference)
import jax
import jax.numpy as jnp
from jax.experimental import pallas as pl


def kernel(x_cheval, x_jockey, x_entraineur, es_cj, ed_cj, es_jc, ed_jc, es_ce, ed_ce, es_ec, ed_ec, c_idx, j_idx, e_idx, linW_c, linb_c, linW_j, linb_j, linW_e, linb_e, Wself0_c, bself0_c, Wself0_j, bself0_j, Wself0_e, bself0_e, Wrel0_cj, Wrel0_jc, Wrel0_ce, Wrel0_ec, Wself1_c, bself1_c, Wself1_j, bself1_j, Wself1_e, bself1_e, Wrel1_cj, Wrel1_jc, Wrel1_ce, Wrel1_ec, clsW1, clsb1, clsW2, clsb2):
    raise NotImplementedError("write your pallas kernel here")


# speedup vs baseline: 1.1610x; 1.1610x over previous
import functools

import jax
import jax.numpy as jnp
from jax import lax
from jax.experimental import pallas as pl
from jax.experimental.pallas import tpu as pltpu
from jax.experimental.pallas import tpu_sc as plsc

N_C, N_J, N_E, D, H, NE, B = 10000, 2000, 2000, 256, 256, 160000, 4096
HH = H // 2
NSUB = 16
EW = 128
NE_PAD = 163840
EROWS = NE_PAD // EW
NPAD_C = 10112
NPAD_S = 2048
ZROWS = NPAD_C

_sc_mesh = plsc.VectorSubcoreMesh(core_axis_name="c", subcore_axis_name="s")


def _agg_body(n_pad, xl_hbm, xr_hbm, es_hbm, ed_hbm, z_hbm,
              out_l, out_r, sidx_v, didx_v, rows_v, s_sh, sem):
  c = lax.axis_index("c")
  s = lax.axis_index("s")
  rpt_z = n_pad // NSUB
  pltpu.sync_copy(z_hbm.at[pl.ds(s * rpt_z, rpt_z)],
                  s_sh.at[pl.ds(s * rpt_z, rpt_z)])
  plsc.subcore_barrier()
  rpt = EROWS // NSUB

  def body(j, carry):
    row = s * rpt + j
    pltpu.sync_copy(es_hbm.at[row], sidx_v)
    pltpu.sync_copy(ed_hbm.at[row], didx_v)

    @pl.when(c == 0)
    def _():
      pltpu.async_copy(xl_hbm.at[sidx_v.at[0]], rows_v, sem).wait()

    @pl.when(c == 1)
    def _():
      pltpu.async_copy(xr_hbm.at[sidx_v.at[0]], rows_v, sem).wait()

    pltpu.sync_copy(rows_v, s_sh.at[didx_v.at[0]], add=True)
    return carry

  lax.fori_loop(0, rpt, body, 0)
  plsc.subcore_barrier()

  @pl.when(c == 0)
  def _():
    pltpu.sync_copy(s_sh.at[pl.ds(s * rpt_z, rpt_z)],
                    out_l.at[pl.ds(s * rpt_z, rpt_z)])

  @pl.when(c == 1)
  def _():
    pltpu.sync_copy(s_sh.at[pl.ds(s * rpt_z, rpt_z)],
                    out_r.at[pl.ds(s * rpt_z, rpt_z)])


@functools.cache
def _make_agg(n_pad):
  return pl.kernel(
      functools.partial(_agg_body, n_pad),
      out_type=(jax.ShapeDtypeStruct((n_pad, HH), jnp.float32),
                jax.ShapeDtypeStruct((n_pad, HH), jnp.float32)),
      mesh=_sc_mesh,
      scratch_types=[
          pltpu.VMEM((1, EW), jnp.int32),
          pltpu.VMEM((1, EW), jnp.int32),
          pltpu.VMEM((EW, HH), jnp.float32),
          pltpu.VMEM_SHARED((n_pad, HH), jnp.float32),
          pltpu.SemaphoreType.DMA,
      ],
  )


def _gather6_body(xcl, xcr, xjl, xjr, xel, xer, cidx, jidx, eidx,
                  ocl, ocr, ojl, ojr, oel, oer, idx_v, rows_v, sem):
  c = lax.axis_index("c")
  s = lax.axis_index("s")
  per_tile = B // NSUB

  def gat(tab, idx_hbm, out):
    def body(j, carry):
      base = s * per_tile + j * EW
      pltpu.sync_copy(idx_hbm.at[pl.ds(base, EW)], idx_v)
      pltpu.async_copy(tab.at[idx_v], rows_v, sem).wait()
      pltpu.sync_copy(rows_v, out.at[pl.ds(base, EW)])
      return carry

    lax.fori_loop(0, per_tile // EW, body, 0)

  @pl.when(c == 0)
  def _():
    gat(xcl, cidx, ocl)
    gat(xjl, jidx, ojl)
    gat(xel, eidx, oel)

  @pl.when(c == 1)
  def _():
    gat(xcr, cidx, ocr)
    gat(xjr, jidx, ojr)
    gat(xer, eidx, oer)


_gather6_kernel = pl.kernel(
    _gather6_body,
    out_type=tuple(jax.ShapeDtypeStruct((B, HH), jnp.float32)
                   for _ in range(6)),
    mesh=_sc_mesh,
    scratch_types=[
        pltpu.VMEM((EW,), jnp.int32),
        pltpu.VMEM((EW, HH), jnp.float32),
        pltpu.SemaphoreType.DMA,
    ],
)


_BR = 400


def _lin_body(x_ref, w_ref, b_ref, ol_ref, or_ref):
  acc = jnp.dot(x_ref[...], w_ref[...], preferred_element_type=jnp.float32)
  acc = jnp.maximum(acc + b_ref[...], 0.0)
  ol_ref[...] = acc[:, :HH]
  or_ref[...] = acc[:, HH:]


def _lin(x, w, b):
  n = x.shape[0]
  return pl.pallas_call(
      _lin_body,
      grid=(n // _BR,),
      in_specs=[
          pl.BlockSpec((_BR, D), lambda i: (i, 0)),
          pl.BlockSpec((D, H), lambda i: (0, 0)),
          pl.BlockSpec((1, H), lambda i: (0, 0)),
      ],
      out_specs=[pl.BlockSpec((_BR, HH), lambda i: (i, 0)),
                 pl.BlockSpec((_BR, HH), lambda i: (i, 0))],
      out_shape=[jax.ShapeDtypeStruct((n, HH), jnp.float32),
                 jax.ShapeDtypeStruct((n, HH), jnp.float32)],
  )(x, w, b.reshape(1, H))


def _mean_mm(sl, sr, cnt, w):
  inv = 1.0 / jnp.maximum(cnt[:, 0:1], 1.0)
  return (jnp.dot(sl * inv, w[:HH, :], preferred_element_type=jnp.float32)
          + jnp.dot(sr * inv, w[HH:, :], preferred_element_type=jnp.float32))


def _conv2_body(xl, xr, sjl, sjr, cj, sel_, ser, ce, ws, b, wj, we, ol, or_):
  acc = (jnp.dot(xl[...], ws[...][:HH, :], preferred_element_type=jnp.float32)
         + jnp.dot(xr[...], ws[...][HH:, :],
                   preferred_element_type=jnp.float32))
  acc += _mean_mm(sjl[...], sjr[...], cj[...], wj[...])
  acc += _mean_mm(sel_[...], ser[...], ce[...], we[...])
  acc = jnp.maximum(acc + b[...], 0.0)
  ol[...] = acc[:, :HH]
  or_[...] = acc[:, HH:]


def _conv2(xl, xr, sjl, sjr, cj, sel_, ser, ce, ws, b, wj, we):
  n = xl.shape[0]
  row = lambda i: (i, 0)
  full = lambda i: (0, 0)
  return pl.pallas_call(
      _conv2_body,
      grid=(n // _BR,),
      in_specs=[
          pl.BlockSpec((_BR, HH), row), pl.BlockSpec((_BR, HH), row),
          pl.BlockSpec((_BR, HH), row), pl.BlockSpec((_BR, HH), row),
          pl.BlockSpec((_BR, 16), row),
          pl.BlockSpec((_BR, HH), row), pl.BlockSpec((_BR, HH), row),
          pl.BlockSpec((_BR, 16), row),
          pl.BlockSpec((H, H), full), pl.BlockSpec((1, H), full),
          pl.BlockSpec((H, H), full), pl.BlockSpec((H, H), full),
      ],
      out_specs=[pl.BlockSpec((_BR, HH), row), pl.BlockSpec((_BR, HH), row)],
      out_shape=[jax.ShapeDtypeStruct((n, HH), jnp.float32),
                 jax.ShapeDtypeStruct((n, HH), jnp.float32)],
  )(xl, xr, sjl, sjr, cj, sel_, ser, ce, ws, b.reshape(1, H), wj, we)


def _conv1_body(xl, xr, sl, sr, cn, ws, b, wr, ol, or_):
  acc = (jnp.dot(xl[...], ws[...][:HH, :], preferred_element_type=jnp.float32)
         + jnp.dot(xr[...], ws[...][HH:, :],
                   preferred_element_type=jnp.float32))
  acc += _mean_mm(sl[...], sr[...], cn[...], wr[...])
  acc = jnp.maximum(acc + b[...], 0.0)
  ol[...] = acc[:, :HH]
  or_[...] = acc[:, HH:]


def _conv1(xl, xr, sl, sr, cn, ws, b, wr):
  n = xl.shape[0]
  row = lambda i: (i, 0)
  full = lambda i: (0, 0)
  return pl.pallas_call(
      _conv1_body,
      grid=(n // _BR,),
      in_specs=[
          pl.BlockSpec((_BR, HH), row), pl.BlockSpec((_BR, HH), row),
          pl.BlockSpec((_BR, HH), row), pl.BlockSpec((_BR, HH), row),
          pl.BlockSpec((_BR, 16), row),
          pl.BlockSpec((H, H), full), pl.BlockSpec((1, H), full),
          pl.BlockSpec((H, H), full),
      ],
      out_specs=[pl.BlockSpec((_BR, HH), row), pl.BlockSpec((_BR, HH), row)],
      out_shape=[jax.ShapeDtypeStruct((n, HH), jnp.float32),
                 jax.ShapeDtypeStruct((n, HH), jnp.float32)],
  )(xl, xr, sl, sr, cn, ws, b.reshape(1, H), wr)


def _cls_body(c0, c1, j0, j1, e0, e1, w1, b1, w2, b2, out):
  w = w1[...]
  acc = jnp.dot(c0[...], w[0:HH, :], preferred_element_type=jnp.float32)
  acc += jnp.dot(c1[...], w[HH:2 * HH, :], preferred_element_type=jnp.float32)
  acc += jnp.dot(j0[...], w[2 * HH:3 * HH, :],
                 preferred_element_type=jnp.float32)
  acc += jnp.dot(j1[...], w[3 * HH:4 * HH, :],
                 preferred_element_type=jnp.float32)
  acc += jnp.dot(e0[...], w[4 * HH:5 * HH, :],
                 preferred_element_type=jnp.float32)
  acc += jnp.dot(e1[...], w[5 * HH:6 * HH, :],
                 preferred_element_type=jnp.float32)
  h = jnp.maximum(acc + b1[...], 0.0)
  out[...] = jnp.dot(h, w2[...], preferred_element_type=jnp.float32) + b2[...]


def _cls(c0, c1, j0, j1, e0, e1, w1, b1, w2, b2):
  br = 512
  row = lambda i: (i, 0)
  full = lambda i: (0, 0)
  return pl.pallas_call(
      _cls_body,
      grid=(B // br,),
      in_specs=[pl.BlockSpec((br, HH), row)] * 6 + [
          pl.BlockSpec((3 * H, H), full), pl.BlockSpec((1, H), full),
          pl.BlockSpec((H, 1), full), pl.BlockSpec((1, 1), full),
      ],
      out_specs=pl.BlockSpec((br, 1), row),
      out_shape=jax.ShapeDtypeStruct((B, 1), jnp.float32),
  )(c0, c1, j0, j1, e0, e1, w1, b1.reshape(1, H), w2, b2.reshape(1, 1))


def _pad_edges(es, ed, n_dst):
  es = jnp.concatenate(
      [es.astype(jnp.int32), jnp.zeros((NE_PAD - NE,), jnp.int32)])
  ed = jnp.concatenate(
      [ed.astype(jnp.int32), jnp.full((NE_PAD - NE,), n_dst, jnp.int32)])
  return es.reshape(EROWS, 1, EW), ed.reshape(EROWS, 1, EW)


def kernel(x_cheval, x_jockey, x_entraineur, es_cj, ed_cj, es_jc, ed_jc,
           es_ce, ed_ce, es_ec, ed_ec, c_idx, j_idx, e_idx, linW_c, linb_c,
           linW_j, linb_j, linW_e, linb_e, Wself0_c, bself0_c, Wself0_j,
           bself0_j, Wself0_e, bself0_e, Wrel0_cj, Wrel0_jc, Wrel0_ce,
           Wrel0_ec, Wself1_c, bself1_c, Wself1_j, bself1_j, Wself1_e,
           bself1_e, Wrel1_cj, Wrel1_jc, Wrel1_ce, Wrel1_ec, clsW1, clsb1,
           clsW2, clsb2):
  z128 = jnp.zeros((ZROWS, HH), jnp.float32)

  es_jc2, ed_jc2 = _pad_edges(es_jc, ed_jc, N_C)
  es_ec2, ed_ec2 = _pad_edges(es_ec, ed_ec, N_C)
  es_cj2, ed_cj2 = _pad_edges(es_cj, ed_cj, N_J)
  es_ce2, ed_ce2 = _pad_edges(es_ce, ed_ce, N_E)

  ones_c = jnp.ones((N_C, HH), jnp.float32)
  ones_s = jnp.ones((N_J, HH), jnp.float32)
  cjc = _make_agg(NPAD_C)(ones_s, ones_s, es_jc2, ed_jc2, z128)[0][:N_C, :16]
  cec = _make_agg(NPAD_C)(ones_s, ones_s, es_ec2, ed_ec2, z128)[0][:N_C, :16]
  ccj = _make_agg(NPAD_S)(ones_c, ones_c, es_cj2, ed_cj2, z128)[0][:N_J, :16]
  cce = _make_agg(NPAD_S)(ones_c, ones_c, es_ce2, ed_ce2, z128)[0][:N_E, :16]

  xcl, xcr = _lin(x_cheval, linW_c, linb_c)
  xjl, xjr = _lin(x_jockey, linW_j, linb_j)
  xel, xer = _lin(x_entraineur, linW_e, linb_e)

  agg_big = _make_agg(NPAD_C)
  agg_small = _make_agg(NPAD_S)
  wrel = {0: (Wrel0_cj, Wrel0_jc, Wrel0_ce, Wrel0_ec),
          1: (Wrel1_cj, Wrel1_jc, Wrel1_ce, Wrel1_ec)}
  wself = {0: (Wself0_c, bself0_c, Wself0_j, bself0_j, Wself0_e, bself0_e),
           1: (Wself1_c, bself1_c, Wself1_j, bself1_j, Wself1_e, bself1_e)}
  for l in (0, 1):
    w_cj, w_jc, w_ce, w_ec = wrel[l]
    ws_c, b_c, ws_j, b_j, ws_e, b_e = wself[l]
    sjc_l, sjc_r = agg_big(xjl, xjr, es_jc2, ed_jc2, z128)
    sec_l, sec_r = agg_big(xel, xer, es_ec2, ed_ec2, z128)
    scj_l, scj_r = agg_small(xcl, xcr, es_cj2, ed_cj2, z128)
    sce_l, sce_r = agg_small(xcl, xcr, es_ce2, ed_ce2, z128)
    sjc_l, sjc_r = sjc_l[:N_C], sjc_r[:N_C]
    sec_l, sec_r = sec_l[:N_C], sec_r[:N_C]
    scj_l, scj_r = scj_l[:N_J], scj_r[:N_J]
    sce_l, sce_r = sce_l[:N_E], sce_r[:N_E]
    ncl, ncr = _conv2(xcl, xcr, sjc_l, sjc_r, cjc, sec_l, sec_r, cec,
                      ws_c, b_c, w_jc, w_ec)
    njl, njr = _conv1(xjl, xjr, scj_l, scj_r, ccj, ws_j, b_j, w_cj)
    nel, ner = _conv1(xel, xer, sce_l, sce_r, cce, ws_e, b_e, w_ce)
    xcl, xcr, xjl, xjr, xel, xer = ncl, ncr, njl, njr, nel, ner

  c0, c1, j0, j1, e0, e1 = _gather6_kernel(
      xcl, xcr, xjl, xjr, xel, xer,
      c_idx.astype(jnp.int32), j_idx.astype(jnp.int32),
      e_idx.astype(jnp.int32))
  return _cls(c0, c1, j0, j1, e0, e1, clsW1, clsb1, clsW2, clsb2)


# speedup vs baseline: 1.5524x; 1.3372x over previous
import functools

import jax
import jax.numpy as jnp
from jax import lax
from jax.experimental import pallas as pl
from jax.experimental.pallas import tpu as pltpu
from jax.experimental.pallas import tpu_sc as plsc

N_C, N_J, N_E, D, H, NE, B = 10000, 2000, 2000, 256, 256, 160000, 4096
HH = H // 2
NSUB = 16
EW = 128
NE_PAD = 163840
EROWS = NE_PAD // EW
NPAD_C = 10112
NPAD_S = 2048
ZROWS = NPAD_C

_sc_mesh = plsc.VectorSubcoreMesh(core_axis_name="c", subcore_axis_name="s")


def _agg_body(n_pad, xl_hbm, xr_hbm, es_hbm, ed_hbm, z_hbm,
              out_l, out_r, idx_s, idx_d, bufs, s_sh, gsem, ssem):
  c = lax.axis_index("c")
  s = lax.axis_index("s")
  rpt_z = n_pad // NSUB
  pltpu.sync_copy(z_hbm.at[pl.ds(s * rpt_z, rpt_z)],
                  s_sh.at[pl.ds(s * rpt_z, rpt_z)])
  plsc.subcore_barrier()
  rpt = EROWS // NSUB
  hr = rpt // 2

  def gather(j, b):
    @pl.when(c == 0)
    def _():
      pltpu.async_copy(xl_hbm.at[idx_s.at[j, 0]], bufs.at[b], gsem)

    @pl.when(c == 1)
    def _():
      pltpu.async_copy(xr_hbm.at[idx_s.at[j, 0]], bufs.at[b], gsem)

  def gwait(b):
    pltpu.make_async_copy(xl_hbm.at[idx_s.at[0, 0]], bufs.at[b], gsem).wait()

  def swait():
    pltpu.make_async_copy(bufs.at[0], s_sh.at[idx_d.at[0, 0]], ssem).wait()

  for half in range(2):
    base = s * rpt + half * hr
    pltpu.sync_copy(es_hbm.at[pl.ds(base, hr)], idx_s)
    pltpu.sync_copy(ed_hbm.at[pl.ds(base, hr)], idx_d)
    gather(0, 0)
    gather(1, 1)

    def body(g, carry):
      for b in range(2):
        j = g * 2 + b
        gwait(b)
        pltpu.async_copy(bufs.at[b], s_sh.at[idx_d.at[j, 0]], ssem, add=True)
        swait()

        @pl.when(g < hr // 2 - 1)
        def _():
          gather(j + 2, b)

      return carry

    lax.fori_loop(0, hr // 2, body, 0)
  plsc.subcore_barrier()

  @pl.when(c == 0)
  def _():
    pltpu.sync_copy(s_sh.at[pl.ds(s * rpt_z, rpt_z)],
                    out_l.at[pl.ds(s * rpt_z, rpt_z)])

  @pl.when(c == 1)
  def _():
    pltpu.sync_copy(s_sh.at[pl.ds(s * rpt_z, rpt_z)],
                    out_r.at[pl.ds(s * rpt_z, rpt_z)])


@functools.cache
def _make_agg(n_pad):
  return pl.kernel(
      functools.partial(_agg_body, n_pad),
      out_type=(jax.ShapeDtypeStruct((n_pad, HH), jnp.float32),
                jax.ShapeDtypeStruct((n_pad, HH), jnp.float32)),
      mesh=_sc_mesh,
      scratch_types=[
          pltpu.VMEM((EROWS // NSUB // 2, 1, EW), jnp.int32),
          pltpu.VMEM((EROWS // NSUB // 2, 1, EW), jnp.int32),
          pltpu.VMEM((2, EW, HH), jnp.float32),
          pltpu.VMEM_SHARED((n_pad, HH), jnp.float32),
          pltpu.SemaphoreType.DMA,
          pltpu.SemaphoreType.DMA,
      ],
  )


def _gather6_body(xcl, xcr, xjl, xjr, xel, xer, cidx, jidx, eidx,
                  ocl, ocr, ojl, ojr, oel, oer, idx_v, rows_v, sem):
  c = lax.axis_index("c")
  s = lax.axis_index("s")
  per_tile = B // NSUB

  def gat(tab, idx_hbm, out):
    def body(j, carry):
      base = s * per_tile + j * EW
      pltpu.sync_copy(idx_hbm.at[pl.ds(base, EW)], idx_v)
      pltpu.async_copy(tab.at[idx_v], rows_v, sem).wait()
      pltpu.sync_copy(rows_v, out.at[pl.ds(base, EW)])
      return carry

    lax.fori_loop(0, per_tile // EW, body, 0)

  @pl.when(c == 0)
  def _():
    gat(xcl, cidx, ocl)
    gat(xjl, jidx, ojl)
    gat(xel, eidx, oel)

  @pl.when(c == 1)
  def _():
    gat(xcr, cidx, ocr)
    gat(xjr, jidx, ojr)
    gat(xer, eidx, oer)


_gather6_kernel = pl.kernel(
    _gather6_body,
    out_type=tuple(jax.ShapeDtypeStruct((B, HH), jnp.float32)
                   for _ in range(6)),
    mesh=_sc_mesh,
    scratch_types=[
        pltpu.VMEM((EW,), jnp.int32),
        pltpu.VMEM((EW, HH), jnp.float32),
        pltpu.SemaphoreType.DMA,
    ],
)


_BR = 400


def _lin_body(x_ref, w_ref, b_ref, ol_ref, or_ref):
  acc = jnp.dot(x_ref[...], w_ref[...], preferred_element_type=jnp.float32)
  acc = jnp.maximum(acc + b_ref[...], 0.0)
  ol_ref[...] = acc[:, :HH]
  or_ref[...] = acc[:, HH:]


def _lin(x, w, b):
  n = x.shape[0]
  return pl.pallas_call(
      _lin_body,
      grid=(n // _BR,),
      in_specs=[
          pl.BlockSpec((_BR, D), lambda i: (i, 0)),
          pl.BlockSpec((D, H), lambda i: (0, 0)),
          pl.BlockSpec((1, H), lambda i: (0, 0)),
      ],
      out_specs=[pl.BlockSpec((_BR, HH), lambda i: (i, 0)),
                 pl.BlockSpec((_BR, HH), lambda i: (i, 0))],
      out_shape=[jax.ShapeDtypeStruct((n, HH), jnp.float32),
                 jax.ShapeDtypeStruct((n, HH), jnp.float32)],
  )(x, w, b.reshape(1, H))


def _mean_mm(sl, sr, cnt, w):
  inv = 1.0 / jnp.maximum(cnt[:, 0:1], 1.0)
  return (jnp.dot(sl * inv, w[:HH, :], preferred_element_type=jnp.float32)
          + jnp.dot(sr * inv, w[HH:, :], preferred_element_type=jnp.float32))


def _conv2_body(xl, xr, sjl, sjr, cj, sel_, ser, ce, ws, b, wj, we, ol, or_):
  acc = (jnp.dot(xl[...], ws[...][:HH, :], preferred_element_type=jnp.float32)
         + jnp.dot(xr[...], ws[...][HH:, :],
                   preferred_element_type=jnp.float32))
  acc += _mean_mm(sjl[...], sjr[...], cj[...], wj[...])
  acc += _mean_mm(sel_[...], ser[...], ce[...], we[...])
  acc = jnp.maximum(acc + b[...], 0.0)
  ol[...] = acc[:, :HH]
  or_[...] = acc[:, HH:]


def _conv2(xl, xr, sjl, sjr, cj, sel_, ser, ce, ws, b, wj, we):
  n = xl.shape[0]
  row = lambda i: (i, 0)
  full = lambda i: (0, 0)
  return pl.pallas_call(
      _conv2_body,
      grid=(n // _BR,),
      in_specs=[
          pl.BlockSpec((_BR, HH), row), pl.BlockSpec((_BR, HH), row),
          pl.BlockSpec((_BR, HH), row), pl.BlockSpec((_BR, HH), row),
          pl.BlockSpec((_BR, 16), row),
          pl.BlockSpec((_BR, HH), row), pl.BlockSpec((_BR, HH), row),
          pl.BlockSpec((_BR, 16), row),
          pl.BlockSpec((H, H), full), pl.BlockSpec((1, H), full),
          pl.BlockSpec((H, H), full), pl.BlockSpec((H, H), full),
      ],
      out_specs=[pl.BlockSpec((_BR, HH), row), pl.BlockSpec((_BR, HH), row)],
      out_shape=[jax.ShapeDtypeStruct((n, HH), jnp.float32),
                 jax.ShapeDtypeStruct((n, HH), jnp.float32)],
  )(xl, xr, sjl, sjr, cj, sel_, ser, ce, ws, b.reshape(1, H), wj, we)


def _conv1_body(xl, xr, sl, sr, cn, ws, b, wr, ol, or_):
  acc = (jnp.dot(xl[...], ws[...][:HH, :], preferred_element_type=jnp.float32)
         + jnp.dot(xr[...], ws[...][HH:, :],
                   preferred_element_type=jnp.float32))
  acc += _mean_mm(sl[...], sr[...], cn[...], wr[...])
  acc = jnp.maximum(acc + b[...], 0.0)
  ol[...] = acc[:, :HH]
  or_[...] = acc[:, HH:]


def _conv1(xl, xr, sl, sr, cn, ws, b, wr):
  n = xl.shape[0]
  row = lambda i: (i, 0)
  full = lambda i: (0, 0)
  return pl.pallas_call(
      _conv1_body,
      grid=(n // _BR,),
      in_specs=[
          pl.BlockSpec((_BR, HH), row), pl.BlockSpec((_BR, HH), row),
          pl.BlockSpec((_BR, HH), row), pl.BlockSpec((_BR, HH), row),
          pl.BlockSpec((_BR, 16), row),
          pl.BlockSpec((H, H), full), pl.BlockSpec((1, H), full),
          pl.BlockSpec((H, H), full),
      ],
      out_specs=[pl.BlockSpec((_BR, HH), row), pl.BlockSpec((_BR, HH), row)],
      out_shape=[jax.ShapeDtypeStruct((n, HH), jnp.float32),
                 jax.ShapeDtypeStruct((n, HH), jnp.float32)],
  )(xl, xr, sl, sr, cn, ws, b.reshape(1, H), wr)


def _cls_body(c0, c1, j0, j1, e0, e1, w1, b1, w2, b2, out):
  w = w1[...]
  acc = jnp.dot(c0[...], w[0:HH, :], preferred_element_type=jnp.float32)
  acc += jnp.dot(c1[...], w[HH:2 * HH, :], preferred_element_type=jnp.float32)
  acc += jnp.dot(j0[...], w[2 * HH:3 * HH, :],
                 preferred_element_type=jnp.float32)
  acc += jnp.dot(j1[...], w[3 * HH:4 * HH, :],
                 preferred_element_type=jnp.float32)
  acc += jnp.dot(e0[...], w[4 * HH:5 * HH, :],
                 preferred_element_type=jnp.float32)
  acc += jnp.dot(e1[...], w[5 * HH:6 * HH, :],
                 preferred_element_type=jnp.float32)
  h = jnp.maximum(acc + b1[...], 0.0)
  out[...] = jnp.dot(h, w2[...], preferred_element_type=jnp.float32) + b2[...]


def _cls(c0, c1, j0, j1, e0, e1, w1, b1, w2, b2):
  br = 512
  row = lambda i: (i, 0)
  full = lambda i: (0, 0)
  return pl.pallas_call(
      _cls_body,
      grid=(B // br,),
      in_specs=[pl.BlockSpec((br, HH), row)] * 6 + [
          pl.BlockSpec((3 * H, H), full), pl.BlockSpec((1, H), full),
          pl.BlockSpec((H, 1), full), pl.BlockSpec((1, 1), full),
      ],
      out_specs=pl.BlockSpec((br, 1), row),
      out_shape=jax.ShapeDtypeStruct((B, 1), jnp.float32),
  )(c0, c1, j0, j1, e0, e1, w1, b1.reshape(1, H), w2, b2.reshape(1, 1))


def _seq(dep, arr):
  return arr + (dep[0, 0] * 0).astype(arr.dtype)


def _pad_edges(es, ed, n_dst):
  es = jnp.concatenate(
      [es.astype(jnp.int32), jnp.zeros((NE_PAD - NE,), jnp.int32)])
  ed = jnp.concatenate(
      [ed.astype(jnp.int32), jnp.full((NE_PAD - NE,), n_dst, jnp.int32)])
  return es.reshape(EROWS, 1, EW), ed.reshape(EROWS, 1, EW)


def kernel(x_cheval, x_jockey, x_entraineur, es_cj, ed_cj, es_jc, ed_jc,
           es_ce, ed_ce, es_ec, ed_ec, c_idx, j_idx, e_idx, linW_c, linb_c,
           linW_j, linb_j, linW_e, linb_e, Wself0_c, bself0_c, Wself0_j,
           bself0_j, Wself0_e, bself0_e, Wrel0_cj, Wrel0_jc, Wrel0_ce,
           Wrel0_ec, Wself1_c, bself1_c, Wself1_j, bself1_j, Wself1_e,
           bself1_e, Wrel1_cj, Wrel1_jc, Wrel1_ce, Wrel1_ec, clsW1, clsb1,
           clsW2, clsb2):
  z128 = jnp.zeros((ZROWS, HH), jnp.float32)

  es_jc2, ed_jc2 = _pad_edges(es_jc, ed_jc, N_C)
  es_ec2, ed_ec2 = _pad_edges(es_ec, ed_ec, N_C)
  es_cj2, ed_cj2 = _pad_edges(es_cj, ed_cj, N_J)
  es_ce2, ed_ce2 = _pad_edges(es_ce, ed_ce, N_E)

  ones_c = jnp.ones((N_C, HH), jnp.float32)
  ones_s = jnp.ones((N_J, HH), jnp.float32)
  cjc_p = _make_agg(NPAD_C)(ones_s, ones_s, es_jc2, ed_jc2, z128)[0]
  cec_p = _make_agg(NPAD_C)(ones_s, ones_s, _seq(cjc_p, es_ec2), ed_ec2,
                            z128)[0]
  ccj_p = _make_agg(NPAD_S)(ones_c, ones_c, _seq(cec_p, es_cj2), ed_cj2,
                            z128)[0]
  cce_p = _make_agg(NPAD_S)(ones_c, ones_c, _seq(ccj_p, es_ce2), ed_ce2,
                            z128)[0]
  cjc, cec = cjc_p[:N_C, :16], cec_p[:N_C, :16]
  ccj, cce = ccj_p[:N_J, :16], cce_p[:N_E, :16]
  dep = cce_p

  xcl, xcr = _lin(x_cheval, linW_c, linb_c)
  xjl, xjr = _lin(x_jockey, linW_j, linb_j)
  xel, xer = _lin(x_entraineur, linW_e, linb_e)

  agg_big = _make_agg(NPAD_C)
  agg_small = _make_agg(NPAD_S)
  wrel = {0: (Wrel0_cj, Wrel0_jc, Wrel0_ce, Wrel0_ec),
          1: (Wrel1_cj, Wrel1_jc, Wrel1_ce, Wrel1_ec)}
  wself = {0: (Wself0_c, bself0_c, Wself0_j, bself0_j, Wself0_e, bself0_e),
           1: (Wself1_c, bself1_c, Wself1_j, bself1_j, Wself1_e, bself1_e)}
  for l in (0, 1):
    w_cj, w_jc, w_ce, w_ec = wrel[l]
    ws_c, b_c, ws_j, b_j, ws_e, b_e = wself[l]
    sjc_l, sjc_r = agg_big(xjl, xjr, _seq(dep, es_jc2), ed_jc2, z128)
    sec_l, sec_r = agg_big(xel, xer, _seq(sjc_l, es_ec2), ed_ec2, z128)
    scj_l, scj_r = agg_small(xcl, xcr, _seq(sec_l, es_cj2), ed_cj2, z128)
    sce_l, sce_r = agg_small(xcl, xcr, _seq(scj_l, es_ce2), ed_ce2, z128)
    dep = sce_l
    sjc_l, sjc_r = sjc_l[:N_C], sjc_r[:N_C]
    sec_l, sec_r = sec_l[:N_C], sec_r[:N_C]
    scj_l, scj_r = scj_l[:N_J], scj_r[:N_J]
    sce_l, sce_r = sce_l[:N_E], sce_r[:N_E]
    ncl, ncr = _conv2(xcl, xcr, sjc_l, sjc_r, cjc, sec_l, sec_r, cec,
                      ws_c, b_c, w_jc, w_ec)
    njl, njr = _conv1(xjl, xjr, scj_l, scj_r, ccj, ws_j, b_j, w_cj)
    nel, ner = _conv1(xel, xer, sce_l, sce_r, cce, ws_e, b_e, w_ce)
    xcl, xcr, xjl, xjr, xel, xer = ncl, ncr, njl, njr, nel, ner

  c0, c1, j0, j1, e0, e1 = _gather6_kernel(
      xcl, xcr, xjl, xjr, xel, xer,
      c_idx.astype(jnp.int32), j_idx.astype(jnp.int32),
      e_idx.astype(jnp.int32))
  return _cls(c0, c1, j0, j1, e0, e1, clsW1, clsb1, clsW2, clsb2)
